# baseline (speedup 1.0000x reference)
import jax
import jax.numpy as jnp
from jax import lax
from jax.experimental import pallas as pl
from jax.experimental.pallas import tpu as pltpu
from jax.experimental.pallas import tpu_sc as plsc

N = 320000
D = 128
S = 10000
NC = 2
NS = 16
NW = NC * NS
R = N // NW
CH = 128
NFULL = R // CH
TAIL = R - NFULL * CH
NB = 3
NMACRO = NFULL // NB
SPW = 624
SREM = S - NS * SPW
CW = 16

_MESH = plsc.VectorSubcoreMesh(
    core_axis_name="c", subcore_axis_name="s", num_cores=NC, num_subcores=NS
)


def _zero_slices(src, dst, sid):
    for o, w in [(0, CH), (CH, CH), (2 * CH, CH), (3 * CH, CH), (4 * CH, 112)]:
        pltpu.sync_copy(src.at[pl.ds(0, w)], dst.at[pl.ds(sid * SPW + o, w)])

    @pl.when(sid == NS - 1)
    def _zero_rem():
        pltpu.sync_copy(src.at[pl.ds(0, SREM)], dst.at[pl.ds(NS * SPW, SREM)])


def _sum_body(data_hbm, seg_hbm, acc_out,
              dbuf, ibuf, itail, dsem, isem, ssem, acc_sh):
    cid = lax.axis_index("c")
    sid = lax.axis_index("s")
    base = (cid * NS + sid) * R

    zeros16 = jnp.zeros((16,), jnp.float32)

    def init_row(r, carry):
        for k in range(D // 16):
            dbuf[0, r, pl.ds(k * 16, 16)] = zeros16
        return carry

    lax.fori_loop(0, CH, init_row, 0)
    _zero_slices(dbuf.at[0], acc_sh, sid)
    plsc.subcore_barrier()

    for b in range(NB):
        off = base + b * CH
        pltpu.async_copy(data_hbm.at[pl.ds(off, CH)], dbuf.at[b], dsem.at[b])
        pltpu.async_copy(seg_hbm.at[pl.ds(off, CH)], ibuf.at[b], isem.at[b])

    def macro(m, carry):
        scats = []
        for b in range(NB):
            off = base + (m * NB + b) * CH
            pltpu.make_async_copy(data_hbm.at[pl.ds(off, CH)],
                                  dbuf.at[b], dsem.at[b]).wait()
            pltpu.make_async_copy(seg_hbm.at[pl.ds(off, CH)],
                                  ibuf.at[b], isem.at[b]).wait()
            scats.append(pltpu.async_copy(
                dbuf.at[b], acc_sh.at[ibuf.at[b]], ssem.at[b], add=True))
        for b in range(NB):
            scats[b].wait()

            @pl.when(m < NMACRO - 1)
            def _refill():
                noff = base + (m * NB + b + NB) * CH
                pltpu.async_copy(data_hbm.at[pl.ds(noff, CH)],
                                 dbuf.at[b], dsem.at[b])
                pltpu.async_copy(seg_hbm.at[pl.ds(noff, CH)],
                                 ibuf.at[b], isem.at[b])
        return carry

    lax.fori_loop(0, NMACRO, macro, 0)

    toff = base + NFULL * CH
    pltpu.sync_copy(data_hbm.at[pl.ds(toff, TAIL)], dbuf.at[0, pl.ds(0, TAIL)])
    pltpu.sync_copy(seg_hbm.at[pl.ds(toff, TAIL)], itail)
    pltpu.sync_copy(dbuf.at[0, pl.ds(0, TAIL)], acc_sh.at[itail], add=True)

    plsc.subcore_barrier()

    r0 = sid * SPW
    pltpu.sync_copy(acc_sh.at[pl.ds(r0, SPW)], acc_out.at[cid, pl.ds(r0, SPW)])

    @pl.when(sid == NS - 1)
    def _read_rem():
        b = NS * SPW
        pltpu.sync_copy(acc_sh.at[pl.ds(b, SREM)], acc_out.at[cid, pl.ds(b, SREM)])


def _cnt_body(seg_hbm, aux_hbm, cnt_out, ibuf, itail, obuf, zbuf,
              isem, ssem, cnt_sh):
    cid = lax.axis_index("c")
    sid = lax.axis_index("s")
    base = (cid * NS + sid) * R

    pltpu.sync_copy(aux_hbm.at[0], obuf)
    pltpu.sync_copy(aux_hbm.at[1], zbuf)
    _zero_slices(zbuf, cnt_sh, sid)
    plsc.subcore_barrier()

    for b in range(NB):
        pltpu.async_copy(seg_hbm.at[pl.ds(base + b * CH, CH)],
                         ibuf.at[b], isem.at[b])

    def macro(m, carry):
        scats = []
        for b in range(NB):
            off = base + (m * NB + b) * CH
            pltpu.make_async_copy(seg_hbm.at[pl.ds(off, CH)],
                                  ibuf.at[b], isem.at[b]).wait()
            scats.append(pltpu.async_copy(
                obuf, cnt_sh.at[ibuf.at[b]], ssem.at[b], add=True))
        for b in range(NB):
            scats[b].wait()

            @pl.when(m < NMACRO - 1)
            def _refill():
                pltpu.async_copy(
                    seg_hbm.at[pl.ds(base + (m * NB + b + NB) * CH, CH)],
                    ibuf.at[b], isem.at[b])
        return carry

    lax.fori_loop(0, NMACRO, macro, 0)

    pltpu.sync_copy(seg_hbm.at[pl.ds(base + NFULL * CH, TAIL)], itail)
    pltpu.sync_copy(obuf.at[pl.ds(0, TAIL)], cnt_sh.at[itail], add=True)

    plsc.subcore_barrier()

    r0 = sid * SPW
    pltpu.sync_copy(cnt_sh.at[pl.ds(r0, SPW)], cnt_out.at[cid, pl.ds(r0, SPW)])

    @pl.when(sid == NS - 1)
    def _read_rem():
        b = NS * SPW
        pltpu.sync_copy(cnt_sh.at[pl.ds(b, SREM)], cnt_out.at[cid, pl.ds(b, SREM)])


def _sc_sums(data, seg):
    return pl.kernel(
        _sum_body,
        out_type=jax.ShapeDtypeStruct((NC, S, D), jnp.float32),
        mesh=_MESH,
        scratch_types=[
            pltpu.VMEM((NB, CH, D), jnp.float32),
            pltpu.VMEM((NB, CH), jnp.int32),
            pltpu.VMEM((TAIL,), jnp.int32),
            pltpu.SemaphoreType.DMA((NB,)),
            pltpu.SemaphoreType.DMA((NB,)),
            pltpu.SemaphoreType.DMA((NB,)),
            pltpu.VMEM_SHARED((S, D), jnp.float32),
        ],
    )(data, seg)


def _sc_counts(seg, aux):
    return pl.kernel(
        _cnt_body,
        out_type=jax.ShapeDtypeStruct((NC, S, CW), jnp.float32),
        mesh=_MESH,
        scratch_types=[
            pltpu.VMEM((NB, CH), jnp.int32),
            pltpu.VMEM((TAIL,), jnp.int32),
            pltpu.VMEM((CH, CW), jnp.float32),
            pltpu.VMEM((CH, CW), jnp.float32),
            pltpu.SemaphoreType.DMA((NB,)),
            pltpu.SemaphoreType.DMA((NB,)),
            pltpu.VMEM_SHARED((S, CW), jnp.float32),
        ],
    )(seg, aux)


def _combine_body(acc_ref, cnt_ref, out_ref):
    sums = acc_ref[0] + acc_ref[1]
    counts = cnt_ref[0, :, 0:1] + cnt_ref[1, :, 0:1]
    out_ref[...] = sums / jnp.maximum(counts, 1.0)


@jax.jit
def kernel(data, segment_ids):
    seg = segment_ids.astype(jnp.int32)
    aux = jnp.concatenate([jnp.ones((1, CH, CW), jnp.float32),
                           jnp.zeros((1, CH, CW), jnp.float32)])
    acc = _sc_sums(data, seg)
    cnt = _sc_counts(seg, aux)
    return pl.pallas_call(
        _combine_body,
        out_shape=jax.ShapeDtypeStruct((S, D), jnp.float32),
    )(acc, cnt)


# speedup vs baseline: 1.0706x; 1.0706x over previous
import jax
import jax.numpy as jnp
from jax import lax
from jax.experimental import pallas as pl
from jax.experimental.pallas import tpu as pltpu
from jax.experimental.pallas import tpu_sc as plsc

N = 320000
D = 128
S = 10000
NC = 2
NS = 16
NW = NC * NS
R = N // NW
CH = 128
NFULL = R // CH
TAIL = R - NFULL * CH
NBS = 2
NMS = NFULL // NBS
NBC = 3
NMC = NFULL // NBC
SPW = 624
SREM = S - NS * SPW
CW = 8

_MESH = plsc.VectorSubcoreMesh(
    core_axis_name="c", subcore_axis_name="s", num_cores=NC, num_subcores=NS
)


def _zero_slices(src, dst, sid):
    for o, w in [(0, CH), (CH, CH), (2 * CH, CH), (3 * CH, CH), (4 * CH, 112)]:
        pltpu.sync_copy(src.at[pl.ds(0, w)], dst.at[pl.ds(sid * SPW + o, w)])

    @pl.when(sid == NS - 1)
    def _zero_rem():
        pltpu.sync_copy(src.at[pl.ds(0, SREM)], dst.at[pl.ds(NS * SPW, SREM)])


def _sum_body(data_hbm, seg_hbm, acc_out,
              dbuf, ibuf, itail, dsem, isem, ssem, acc_sh):
    cid = lax.axis_index("c")
    sid = lax.axis_index("s")
    base = (cid * NS + sid) * R

    zeros16 = jnp.zeros((16,), jnp.float32)

    def init_row(r, carry):
        for k in range(D // 16):
            dbuf[0, r, pl.ds(k * 16, 16)] = zeros16
        return carry

    lax.fori_loop(0, CH, init_row, 0)
    _zero_slices(dbuf.at[0], acc_sh, sid)
    plsc.subcore_barrier()

    for b in range(NBS):
        off = base + b * CH
        pltpu.async_copy(data_hbm.at[pl.ds(off, CH)], dbuf.at[b], dsem.at[b])
        pltpu.async_copy(seg_hbm.at[pl.ds(off, CH)], ibuf.at[b], isem.at[b])

    def macro(m, carry):
        for b in range(NBS):
            off = base + (m * NBS + b) * CH
            pltpu.make_async_copy(data_hbm.at[pl.ds(off, CH)],
                                  dbuf.at[b], dsem.at[b]).wait()
            pltpu.make_async_copy(seg_hbm.at[pl.ds(off, CH)],
                                  ibuf.at[b], isem.at[b]).wait()
            pltpu.async_copy(dbuf.at[b], acc_sh.at[ibuf.at[b]], ssem.at[b],
                             add=True).wait()

            @pl.when(m < NMS - 1)
            def _refill():
                noff = off + NBS * CH
                pltpu.async_copy(data_hbm.at[pl.ds(noff, CH)],
                                 dbuf.at[b], dsem.at[b])
                pltpu.async_copy(seg_hbm.at[pl.ds(noff, CH)],
                                 ibuf.at[b], isem.at[b])
        return carry

    lax.fori_loop(0, NMS, macro, 0)

    toff = base + NFULL * CH
    pltpu.sync_copy(data_hbm.at[pl.ds(toff, TAIL)], dbuf.at[0, pl.ds(0, TAIL)])
    pltpu.sync_copy(seg_hbm.at[pl.ds(toff, TAIL)], itail)
    pltpu.sync_copy(dbuf.at[0, pl.ds(0, TAIL)], acc_sh.at[itail], add=True)

    plsc.subcore_barrier()

    r0 = sid * SPW
    pltpu.sync_copy(acc_sh.at[pl.ds(r0, SPW)], acc_out.at[cid, pl.ds(r0, SPW)])

    @pl.when(sid == NS - 1)
    def _read_rem():
        b = NS * SPW
        pltpu.sync_copy(acc_sh.at[pl.ds(b, SREM)], acc_out.at[cid, pl.ds(b, SREM)])


def _cnt_body(seg_hbm, aux_hbm, cnt_out, ibuf, itail, obuf, zbuf,
              isem, ssem, cnt_sh):
    cid = lax.axis_index("c")
    sid = lax.axis_index("s")
    base = (cid * NS + sid) * R

    pltpu.sync_copy(aux_hbm.at[0], obuf)
    pltpu.sync_copy(aux_hbm.at[1], zbuf)
    _zero_slices(zbuf, cnt_sh, sid)
    plsc.subcore_barrier()

    for b in range(NBC):
        pltpu.async_copy(seg_hbm.at[pl.ds(base + b * CH, CH)],
                         ibuf.at[b], isem.at[b])

    def macro(m, carry):
        scats = []
        for b in range(NBC):
            off = base + (m * NBC + b) * CH
            pltpu.make_async_copy(seg_hbm.at[pl.ds(off, CH)],
                                  ibuf.at[b], isem.at[b]).wait()
            scats.append(pltpu.async_copy(
                obuf, cnt_sh.at[ibuf.at[b]], ssem.at[b], add=True))
        for b in range(NBC):
            scats[b].wait()

            @pl.when(m < NMC - 1)
            def _refill():
                pltpu.async_copy(
                    seg_hbm.at[pl.ds(base + (m * NBC + b + NBC) * CH, CH)],
                    ibuf.at[b], isem.at[b])
        return carry

    lax.fori_loop(0, NMC, macro, 0)

    pltpu.sync_copy(seg_hbm.at[pl.ds(base + NFULL * CH, TAIL)], itail)
    pltpu.sync_copy(obuf.at[pl.ds(0, TAIL)], cnt_sh.at[itail], add=True)

    plsc.subcore_barrier()

    r0 = sid * SPW
    pltpu.sync_copy(cnt_sh.at[pl.ds(r0, SPW)], cnt_out.at[cid, pl.ds(r0, SPW)])

    @pl.when(sid == NS - 1)
    def _read_rem():
        b = NS * SPW
        pltpu.sync_copy(cnt_sh.at[pl.ds(b, SREM)], cnt_out.at[cid, pl.ds(b, SREM)])


def _sc_sums(data, seg):
    return pl.kernel(
        _sum_body,
        out_type=jax.ShapeDtypeStruct((NC, S, D), jnp.float32),
        mesh=_MESH,
        scratch_types=[
            pltpu.VMEM((NBS, CH, D), jnp.float32),
            pltpu.VMEM((NBS, CH), jnp.int32),
            pltpu.VMEM((TAIL,), jnp.int32),
            pltpu.SemaphoreType.DMA((NBS,)),
            pltpu.SemaphoreType.DMA((NBS,)),
            pltpu.SemaphoreType.DMA((NBS,)),
            pltpu.VMEM_SHARED((S, D), jnp.float32),
        ],
    )(data, seg)


def _sc_counts(seg, aux):
    return pl.kernel(
        _cnt_body,
        out_type=jax.ShapeDtypeStruct((NC, S, CW), jnp.float32),
        mesh=_MESH,
        scratch_types=[
            pltpu.VMEM((NBC, CH), jnp.int32),
            pltpu.VMEM((TAIL,), jnp.int32),
            pltpu.VMEM((CH, CW), jnp.float32),
            pltpu.VMEM((CH, CW), jnp.float32),
            pltpu.SemaphoreType.DMA((NBC,)),
            pltpu.SemaphoreType.DMA((NBC,)),
            pltpu.VMEM_SHARED((S, CW), jnp.float32),
        ],
    )(seg, aux)


def _combine_body(acc_ref, cnt_ref, out_ref):
    sums = acc_ref[0] + acc_ref[1]
    counts = cnt_ref[0, :, 0:1] + cnt_ref[1, :, 0:1]
    out_ref[...] = sums / jnp.maximum(counts, 1.0)


@jax.jit
def kernel(data, segment_ids):
    seg = segment_ids.astype(jnp.int32)
    aux = jnp.concatenate([jnp.ones((1, CH, CW), jnp.float32),
                           jnp.zeros((1, CH, CW), jnp.float32)])
    acc = _sc_sums(data, seg)
    cnt = _sc_counts(seg, aux)
    return pl.pallas_call(
        _combine_body,
        out_shape=jax.ShapeDtypeStruct((S, D), jnp.float32),
    )(acc, cnt)


# speedup vs baseline: 1.2915x; 1.2064x over previous
import jax
import jax.numpy as jnp
from jax import lax
from jax.experimental import pallas as pl
from jax.experimental.pallas import tpu as pltpu
from jax.experimental.pallas import tpu_sc as plsc

N = 320000
D = 128
S = 10000
NC = 2
NS = 16
NW = NC * NS
R = N // NW
CH = 128
NFULL = R // CH
TAIL = R - NFULL * CH
NBS = 2
NMS = NFULL // NBS
SPW = 624
SREM = S - NS * SPW

_MESH = plsc.VectorSubcoreMesh(
    core_axis_name="c", subcore_axis_name="s", num_cores=NC, num_subcores=NS
)


def _vector_counts(cnt_local, iv):
    cnt, last = plsc.scan_count(iv)
    plsc.addupdate_scatter(cnt_local, [iv], cnt.astype(jnp.float32), mask=last)


def _sum_body(data_hbm, seg_hbm, acc_out, cnt_out,
              dbuf, ibuf, itail, cnt_local, dsem, isem, ssem, acc_sh):
    cid = lax.axis_index("c")
    sid = lax.axis_index("s")
    wid = cid * NS + sid
    base = wid * R

    zeros16 = jnp.zeros((16,), jnp.float32)

    def init_row(r, carry):
        for k in range(D // 16):
            dbuf[0, r, pl.ds(k * 16, 16)] = zeros16
        return carry

    lax.fori_loop(0, CH, init_row, 0)

    def init_cnt(j, carry):
        cnt_local[pl.ds(j * 16, 16)] = zeros16
        return carry

    lax.fori_loop(0, S // 16, init_cnt, 0)

    for o, w in [(0, CH), (CH, CH), (2 * CH, CH), (3 * CH, CH), (4 * CH, 112)]:
        pltpu.sync_copy(dbuf.at[0, pl.ds(0, w)],
                        acc_sh.at[pl.ds(sid * SPW + o, w)])

    @pl.when(sid == NS - 1)
    def _zero_rem():
        pltpu.sync_copy(dbuf.at[0, pl.ds(0, SREM)],
                        acc_sh.at[pl.ds(NS * SPW, SREM)])

    plsc.subcore_barrier()

    for b in range(NBS):
        off = base + b * CH
        pltpu.async_copy(data_hbm.at[pl.ds(off, CH)], dbuf.at[b], dsem.at[b])
        pltpu.async_copy(seg_hbm.at[pl.ds(off, CH)], ibuf.at[b], isem.at[b])

    def macro(m, carry):
        for b in range(NBS):
            off = base + (m * NBS + b) * CH
            pltpu.make_async_copy(data_hbm.at[pl.ds(off, CH)],
                                  dbuf.at[b], dsem.at[b]).wait()
            pltpu.make_async_copy(seg_hbm.at[pl.ds(off, CH)],
                                  ibuf.at[b], isem.at[b]).wait()
            scat = pltpu.async_copy(dbuf.at[b], acc_sh.at[ibuf.at[b]],
                                    ssem.at[b], add=True)
            for k in range(CH // 16):
                _vector_counts(cnt_local, ibuf[b, pl.ds(k * 16, 16)])
            scat.wait()

            @pl.when(m < NMS - 1)
            def _refill():
                noff = off + NBS * CH
                pltpu.async_copy(data_hbm.at[pl.ds(noff, CH)],
                                 dbuf.at[b], dsem.at[b])
                pltpu.async_copy(seg_hbm.at[pl.ds(noff, CH)],
                                 ibuf.at[b], isem.at[b])
        return carry

    lax.fori_loop(0, NMS, macro, 0)

    toff = base + NFULL * CH
    pltpu.sync_copy(data_hbm.at[pl.ds(toff, TAIL)], dbuf.at[0, pl.ds(0, TAIL)])
    pltpu.sync_copy(seg_hbm.at[pl.ds(toff, TAIL)], itail)
    pltpu.sync_copy(dbuf.at[0, pl.ds(0, TAIL)], acc_sh.at[itail], add=True)
    _vector_counts(cnt_local, itail[...])

    pltpu.sync_copy(cnt_local, cnt_out.at[pl.ds(wid * S, S)])

    plsc.subcore_barrier()

    r0 = sid * SPW
    pltpu.sync_copy(acc_sh.at[pl.ds(r0, SPW)], acc_out.at[cid, pl.ds(r0, SPW)])

    @pl.when(sid == NS - 1)
    def _read_rem():
        b = NS * SPW
        pltpu.sync_copy(acc_sh.at[pl.ds(b, SREM)], acc_out.at[cid, pl.ds(b, SREM)])


def _sc_sums_counts(data, seg):
    return pl.kernel(
        _sum_body,
        out_type=(
            jax.ShapeDtypeStruct((NC, S, D), jnp.float32),
            jax.ShapeDtypeStruct((NW * S,), jnp.float32),
        ),
        mesh=_MESH,
        compiler_params=pltpu.CompilerParams(needs_layout_passes=False),
        scratch_types=[
            pltpu.VMEM((NBS, CH, D), jnp.float32),
            pltpu.VMEM((NBS, CH), jnp.int32),
            pltpu.VMEM((TAIL,), jnp.int32),
            pltpu.VMEM((S,), jnp.float32),
            pltpu.SemaphoreType.DMA((NBS,)),
            pltpu.SemaphoreType.DMA((NBS,)),
            pltpu.SemaphoreType.DMA((NBS,)),
            pltpu.VMEM_SHARED((S, D), jnp.float32),
        ],
    )(data, seg)


def _combine_body(acc_ref, cnt_ref, out_ref):
    sums = acc_ref[0] + acc_ref[1]
    counts = jnp.sum(cnt_ref[...], axis=1, keepdims=True)
    out_ref[...] = sums / jnp.maximum(counts, 1.0)


@jax.jit
def kernel(data, segment_ids):
    seg = segment_ids.astype(jnp.int32)
    acc, cnt = _sc_sums_counts(data, seg)
    cnt_t = cnt.reshape(NW, S).T
    return pl.pallas_call(
        _combine_body,
        out_shape=jax.ShapeDtypeStruct((S, D), jnp.float32),
    )(acc, cnt_t)


# speedup vs baseline: 1.3480x; 1.0437x over previous
import jax
import jax.numpy as jnp
from jax import lax
from jax.experimental import pallas as pl
from jax.experimental.pallas import tpu as pltpu
from jax.experimental.pallas import tpu_sc as plsc

N = 320000
D = 128
S = 10000
NC = 2
NS = 16
NW = NC * NS
R = N // NW
CH = 128
NFULL = R // CH
TAIL = R - NFULL * CH
NBS = 2
NMS = NFULL // NBS
SPW = 624
SREM = S - NS * SPW

_MESH = plsc.VectorSubcoreMesh(
    core_axis_name="c", subcore_axis_name="s", num_cores=NC, num_subcores=NS
)


def _vector_counts(cnt_local, iv):
    cnt, last = plsc.scan_count(iv)
    plsc.addupdate_scatter(cnt_local, [iv], cnt.astype(jnp.float32), mask=last)


def _sum_body(data_hbm, seg_hbm, acc_out, cnt_out,
              dbuf, ibuf, itail, cnt_local, dsem, isem, ssem, acc_sh):
    cid = lax.axis_index("c")
    sid = lax.axis_index("s")
    wid = cid * NS + sid
    base = wid * R

    zeros16 = jnp.zeros((16,), jnp.float32)

    for b in range(1, NBS):
        off = base + b * CH
        pltpu.async_copy(data_hbm.at[pl.ds(off, CH)], dbuf.at[b], dsem.at[b])
        pltpu.async_copy(seg_hbm.at[pl.ds(off, CH)], ibuf.at[b], isem.at[b])

    def init_row(r, carry):
        for k in range(D // 16):
            dbuf[0, r, pl.ds(k * 16, 16)] = zeros16
        return carry

    lax.fori_loop(0, CH, init_row, 0)

    zcopies = [
        pltpu.async_copy(dbuf.at[0, pl.ds(0, w)],
                         acc_sh.at[pl.ds(sid * SPW + o, w)], ssem.at[0])
        for o, w in [(0, CH), (CH, CH), (2 * CH, CH), (3 * CH, CH),
                     (4 * CH, 112)]
    ]

    @pl.when(sid == NS - 1)
    def _zero_rem():
        pltpu.sync_copy(dbuf.at[0, pl.ds(0, SREM)],
                        acc_sh.at[pl.ds(NS * SPW, SREM)])

    def init_cnt(j, carry):
        cnt_local[pl.ds(j * 16, 16)] = zeros16
        return carry

    lax.fori_loop(0, S // 16, init_cnt, 0)

    for zc in zcopies:
        zc.wait()

    pltpu.async_copy(data_hbm.at[pl.ds(base, CH)], dbuf.at[0], dsem.at[0])
    pltpu.async_copy(seg_hbm.at[pl.ds(base, CH)], ibuf.at[0], isem.at[0])

    plsc.subcore_barrier()

    def macro(m, carry):
        for b in range(NBS):
            off = base + (m * NBS + b) * CH
            pltpu.make_async_copy(data_hbm.at[pl.ds(off, CH)],
                                  dbuf.at[b], dsem.at[b]).wait()
            pltpu.make_async_copy(seg_hbm.at[pl.ds(off, CH)],
                                  ibuf.at[b], isem.at[b]).wait()
            scat = pltpu.async_copy(dbuf.at[b], acc_sh.at[ibuf.at[b]],
                                    ssem.at[b], add=True)
            for k in range(CH // 16):
                _vector_counts(cnt_local, ibuf[b, pl.ds(k * 16, 16)])
            scat.wait()

            @pl.when(m < NMS - 1)
            def _refill():
                noff = off + NBS * CH
                pltpu.async_copy(data_hbm.at[pl.ds(noff, CH)],
                                 dbuf.at[b], dsem.at[b])
                pltpu.async_copy(seg_hbm.at[pl.ds(noff, CH)],
                                 ibuf.at[b], isem.at[b])
        return carry

    lax.fori_loop(0, NMS, macro, 0)

    toff = base + NFULL * CH
    pltpu.sync_copy(data_hbm.at[pl.ds(toff, TAIL)], dbuf.at[0, pl.ds(0, TAIL)])
    pltpu.sync_copy(seg_hbm.at[pl.ds(toff, TAIL)], itail)
    pltpu.sync_copy(dbuf.at[0, pl.ds(0, TAIL)], acc_sh.at[itail], add=True)
    _vector_counts(cnt_local, itail[...])

    pltpu.sync_copy(cnt_local, cnt_out.at[pl.ds(wid * S, S)])

    plsc.subcore_barrier()

    r0 = sid * SPW
    pltpu.sync_copy(acc_sh.at[pl.ds(r0, SPW)], acc_out.at[cid, pl.ds(r0, SPW)])

    @pl.when(sid == NS - 1)
    def _read_rem():
        b = NS * SPW
        pltpu.sync_copy(acc_sh.at[pl.ds(b, SREM)], acc_out.at[cid, pl.ds(b, SREM)])


def _sc_sums_counts(data, seg):
    return pl.kernel(
        _sum_body,
        out_type=(
            jax.ShapeDtypeStruct((NC, S, D), jnp.float32),
            jax.ShapeDtypeStruct((NW * S,), jnp.float32),
        ),
        mesh=_MESH,
        compiler_params=pltpu.CompilerParams(needs_layout_passes=False),
        scratch_types=[
            pltpu.VMEM((NBS, CH, D), jnp.float32),
            pltpu.VMEM((NBS, CH), jnp.int32),
            pltpu.VMEM((TAIL,), jnp.int32),
            pltpu.VMEM((S,), jnp.float32),
            pltpu.SemaphoreType.DMA((NBS,)),
            pltpu.SemaphoreType.DMA((NBS,)),
            pltpu.SemaphoreType.DMA((NBS,)),
            pltpu.VMEM_SHARED((S, D), jnp.float32),
        ],
    )(data, seg)


def _combine_body(acc_ref, cnt_ref, out_ref):
    sums = acc_ref[0] + acc_ref[1]
    ones = jnp.ones((NW, 1), jnp.float32)
    counts = lax.dot_general(cnt_ref[...], ones, (((0,), (0,)), ((), ())),
                             preferred_element_type=jnp.float32)
    out_ref[...] = sums / jnp.maximum(counts, 1.0)


@jax.jit
def kernel(data, segment_ids):
    seg = segment_ids.astype(jnp.int32)
    acc, cnt = _sc_sums_counts(data, seg)
    return pl.pallas_call(
        _combine_body,
        out_shape=jax.ShapeDtypeStruct((S, D), jnp.float32),
    )(acc, cnt.reshape(NW, S))


# speedup vs baseline: 1.4245x; 1.0567x over previous
import jax
import jax.numpy as jnp
from jax import lax
from jax.experimental import pallas as pl
from jax.experimental.pallas import tpu as pltpu
from jax.experimental.pallas import tpu_sc as plsc

N = 320000
D = 128
S = 10000
NC = 2
NS = 16
NW = NC * NS
R = N // NW
CH = 64
NFULL = R // CH
TAIL = R - NFULL * CH
NB = 4
NM = NFULL // NB
SPW = 624
SREM = S - NS * SPW
ZSL = NB - 1

_MESH = plsc.VectorSubcoreMesh(
    core_axis_name="c", subcore_axis_name="s", num_cores=NC, num_subcores=NS
)


def _vector_counts(cnt_local, iv):
    cnt, last = plsc.scan_count(iv)
    plsc.addupdate_scatter(cnt_local, [iv], cnt.astype(jnp.float32), mask=last)


def _sum_body(data_hbm, seg_hbm, acc_out, cnt_out,
              dbuf, ibuf, itail, cnt_local, dsem, isem, ssem, acc_sh):
    cid = lax.axis_index("c")
    sid = lax.axis_index("s")
    wid = cid * NS + sid
    base = wid * R

    zeros16 = jnp.zeros((16,), jnp.float32)

    for b in range(ZSL):
        off = base + b * CH
        pltpu.async_copy(data_hbm.at[pl.ds(off, CH)], dbuf.at[b], dsem.at[b])
        pltpu.async_copy(seg_hbm.at[pl.ds(off, CH)], ibuf.at[b], isem.at[b])

    def init_row(r, carry):
        for k in range(D // 16):
            dbuf[ZSL, r, pl.ds(k * 16, 16)] = zeros16
        return carry

    lax.fori_loop(0, CH, init_row, 0)

    zcopies = [
        pltpu.async_copy(dbuf.at[ZSL, pl.ds(0, w)],
                         acc_sh.at[pl.ds(sid * SPW + o, w)], ssem.at[ZSL])
        for o, w in [(0, 64), (64, 64), (128, 64), (192, 64), (256, 64),
                     (320, 64), (384, 64), (448, 64), (512, 64), (576, 48)]
    ]

    @pl.when(sid == NS - 1)
    def _zero_rem():
        pltpu.sync_copy(dbuf.at[ZSL, pl.ds(0, SREM)],
                        acc_sh.at[pl.ds(NS * SPW, SREM)])

    def init_cnt(j, carry):
        cnt_local[pl.ds(j * 16, 16)] = zeros16
        return carry

    lax.fori_loop(0, S // 16, init_cnt, 0)

    for zc in zcopies:
        zc.wait()

    zoff = base + ZSL * CH
    pltpu.async_copy(data_hbm.at[pl.ds(zoff, CH)], dbuf.at[ZSL], dsem.at[ZSL])
    pltpu.async_copy(seg_hbm.at[pl.ds(zoff, CH)], ibuf.at[ZSL], isem.at[ZSL])

    plsc.subcore_barrier()

    def macro(m, carry):
        for b in range(NB):
            c = m * NB + b
            off = base + c * CH
            pltpu.make_async_copy(data_hbm.at[pl.ds(off, CH)],
                                  dbuf.at[b], dsem.at[b]).wait()
            pltpu.make_async_copy(seg_hbm.at[pl.ds(off, CH)],
                                  ibuf.at[b], isem.at[b]).wait()
            for k in range(CH // 16):
                _vector_counts(cnt_local, ibuf[b, pl.ds(k * 16, 16)])
            pltpu.async_copy(dbuf.at[b], acc_sh.at[ibuf.at[b]], ssem.at[b],
                             add=True)

            prev = (b + NB - 1) % NB

            @pl.when(c >= 1)
            def _retire_prev():
                pltpu.make_async_copy(dbuf.at[prev], acc_sh.at[ibuf.at[prev]],
                                      ssem.at[prev]).wait()

            @pl.when((c >= 1) & (c + NB - 1 < NFULL))
            def _refill_prev():
                noff = base + (c + NB - 1) * CH
                pltpu.async_copy(data_hbm.at[pl.ds(noff, CH)],
                                 dbuf.at[prev], dsem.at[prev])
                pltpu.async_copy(seg_hbm.at[pl.ds(noff, CH)],
                                 ibuf.at[prev], isem.at[prev])
        return carry

    lax.fori_loop(0, NM, macro, 0)

    pltpu.make_async_copy(dbuf.at[NB - 1], acc_sh.at[ibuf.at[NB - 1]],
                          ssem.at[NB - 1]).wait()

    toff = base + NFULL * CH
    pltpu.sync_copy(data_hbm.at[pl.ds(toff, TAIL)], dbuf.at[0, pl.ds(0, TAIL)])
    pltpu.sync_copy(seg_hbm.at[pl.ds(toff, TAIL)], itail)
    pltpu.sync_copy(dbuf.at[0, pl.ds(0, TAIL)], acc_sh.at[itail], add=True)
    _vector_counts(cnt_local, itail[...])

    pltpu.sync_copy(cnt_local, cnt_out.at[pl.ds(wid * S, S)])

    plsc.subcore_barrier()

    r0 = sid * SPW
    pltpu.sync_copy(acc_sh.at[pl.ds(r0, SPW)], acc_out.at[cid, pl.ds(r0, SPW)])

    @pl.when(sid == NS - 1)
    def _read_rem():
        b = NS * SPW
        pltpu.sync_copy(acc_sh.at[pl.ds(b, SREM)], acc_out.at[cid, pl.ds(b, SREM)])


def _sc_sums_counts(data, seg):
    return pl.kernel(
        _sum_body,
        out_type=(
            jax.ShapeDtypeStruct((NC, S, D), jnp.float32),
            jax.ShapeDtypeStruct((NW * S,), jnp.float32),
        ),
        mesh=_MESH,
        compiler_params=pltpu.CompilerParams(needs_layout_passes=False),
        scratch_types=[
            pltpu.VMEM((NB, CH, D), jnp.float32),
            pltpu.VMEM((NB, CH), jnp.int32),
            pltpu.VMEM((TAIL,), jnp.int32),
            pltpu.VMEM((S,), jnp.float32),
            pltpu.SemaphoreType.DMA((NB,)),
            pltpu.SemaphoreType.DMA((NB,)),
            pltpu.SemaphoreType.DMA((NB,)),
            pltpu.VMEM_SHARED((S, D), jnp.float32),
        ],
    )(data, seg)


def _combine_body(acc_ref, cnt_ref, out_ref):
    sums = acc_ref[0] + acc_ref[1]
    ones = jnp.ones((NW, 1), jnp.float32)
    counts = lax.dot_general(cnt_ref[...], ones, (((0,), (0,)), ((), ())),
                             preferred_element_type=jnp.float32)
    out_ref[...] = sums / jnp.maximum(counts, 1.0)


@jax.jit
def kernel(data, segment_ids):
    seg = segment_ids.astype(jnp.int32)
    acc, cnt = _sc_sums_counts(data, seg)
    return pl.pallas_call(
        _combine_body,
        out_shape=jax.ShapeDtypeStruct((S, D), jnp.float32),
    )(acc, cnt.reshape(NW, S))
